# baseline (speedup 1.0000x reference)
import functools

import jax
import jax.numpy as jnp
from jax import lax
from jax.experimental import pallas as pl
from jax.experimental.pallas import tpu as pltpu
from jax.experimental.pallas import tpu_sc as plsc

B = 16384
MF_DIM = 16
MLP_DIM = 32

_NC = 2
_NS = 16
_NW = _NC * _NS
_BPW = B // _NW
_CH = 128
_NCH = _BPW // _CH

_BLK = 2048


def _sc_gather(uidx2, iidx2, mf_user, mf_item, mlp_user, mlp_item):
    mesh = plsc.VectorSubcoreMesh(core_axis_name="c", subcore_axis_name="s")

    @functools.partial(
        pl.kernel,
        mesh=mesh,
        compiler_params=pltpu.CompilerParams(use_tc_tiling_on_sc=False),
        out_type=[
            jax.ShapeDtypeStruct((B, MLP_DIM), jnp.float32),
            jax.ShapeDtypeStruct((B, MLP_DIM), jnp.float32),
            jax.ShapeDtypeStruct((B, MF_DIM), jnp.float32),
            jax.ShapeDtypeStruct((B, MF_DIM), jnp.float32),
        ],
        scratch_types=[
            pltpu.VMEM((_NCH, _CH), jnp.int32),
            pltpu.VMEM((_NCH, _CH), jnp.int32),
            pltpu.VMEM((_BPW, MLP_DIM), jnp.float32),
            pltpu.VMEM((_BPW, MLP_DIM), jnp.float32),
            pltpu.VMEM((_BPW, MF_DIM), jnp.float32),
            pltpu.VMEM((_BPW, MF_DIM), jnp.float32),
            pltpu.SemaphoreType.DMA,
        ],
    )
    def k(uidx_hbm, iidx_hbm, mfu_hbm, mfi_hbm, mlpu_hbm, mlpi_hbm,
          out_mlpu, out_mlpi, out_mfu, out_mfi,
          uidx_v, iidx_v, mlpu_v, mlpi_v, mfu_v, mfi_v, sem):
        wid = lax.axis_index("s") * _NC + lax.axis_index("c")
        row = wid * _NCH
        pltpu.sync_copy(uidx_hbm.at[pl.ds(row, _NCH)], uidx_v)
        pltpu.sync_copy(iidx_hbm.at[pl.ds(row, _NCH)], iidx_v)
        copies = []
        for j in range(_NCH):
            sl = pl.ds(j * _CH, _CH)
            copies.append(pltpu.async_copy(mlpu_hbm.at[uidx_v.at[j]], mlpu_v.at[sl], sem))
            copies.append(pltpu.async_copy(mlpi_hbm.at[iidx_v.at[j]], mlpi_v.at[sl], sem))
            copies.append(pltpu.async_copy(mfu_hbm.at[uidx_v.at[j]], mfu_v.at[sl], sem))
            copies.append(pltpu.async_copy(mfi_hbm.at[iidx_v.at[j]], mfi_v.at[sl], sem))
        for c in copies:
            c.wait()
        bs = pl.ds(wid * _BPW, _BPW)
        pltpu.sync_copy(mlpu_v, out_mlpu.at[bs])
        pltpu.sync_copy(mlpi_v, out_mlpi.at[bs])
        pltpu.sync_copy(mfu_v, out_mfu.at[bs])
        pltpu.sync_copy(mfi_v, out_mfi.at[bs])

    return k(uidx2, iidx2, mf_user, mf_item, mlp_user, mlp_item)


def _mlp_body(mlpu_ref, mlpi_ref, mfu_ref, mfi_ref,
              w1a_ref, w1b_ref, b1_ref, w2_ref, b2_ref,
              wph_ref, wpm_ref, bp_ref, out_ref):
    h = jnp.dot(mlpu_ref[...], w1a_ref[...], preferred_element_type=jnp.float32)
    h = h + jnp.dot(mlpi_ref[...], w1b_ref[...], preferred_element_type=jnp.float32)
    h = jnp.maximum(h + b1_ref[...], 0.0)
    h2 = jnp.dot(h, w2_ref[...], preferred_element_type=jnp.float32)
    h2 = jnp.maximum(h2 + b2_ref[...], 0.0)
    mf = mfu_ref[...] * mfi_ref[...]
    logit = jnp.sum(h2 * wph_ref[...] + mf * wpm_ref[...], axis=1, keepdims=True)
    out_ref[...] = jax.nn.sigmoid(logit + bp_ref[...])[:, 0]


def _mlp_call(mlpu, mlpi, mfu, mfi, w1a, w1b, b1, w2, b2, wph, wpm, bp):
    full = lambda shape: pl.BlockSpec(shape, lambda i: (0, 0))
    return pl.pallas_call(
        _mlp_body,
        grid=(B // _BLK,),
        in_specs=[
            pl.BlockSpec((_BLK, MLP_DIM), lambda i: (i, 0)),
            pl.BlockSpec((_BLK, MLP_DIM), lambda i: (i, 0)),
            pl.BlockSpec((_BLK, MF_DIM), lambda i: (i, 0)),
            pl.BlockSpec((_BLK, MF_DIM), lambda i: (i, 0)),
            full((MLP_DIM, MLP_DIM)),
            full((MLP_DIM, MLP_DIM)),
            full((1, MLP_DIM)),
            full((MLP_DIM, MF_DIM)),
            full((1, MF_DIM)),
            full((1, MF_DIM)),
            full((1, MF_DIM)),
            full((1, 1)),
        ],
        out_specs=pl.BlockSpec((_BLK,), lambda i: (i,)),
        out_shape=jax.ShapeDtypeStruct((B,), jnp.float32),
    )(mlpu, mlpi, mfu, mfi, w1a, w1b, b1, w2, b2, wph, wpm, bp)


def kernel(user_input, item_input, mf_user, mf_item, mlp_user, mlp_item,
           W1, b1, W2, b2, Wp, bp):
    uidx2 = user_input.astype(jnp.int32).reshape(B // _CH, _CH)
    iidx2 = item_input.astype(jnp.int32).reshape(B // _CH, _CH)
    mlpu, mlpi, mfu, mfi = _sc_gather(uidx2, iidx2, mf_user, mf_item,
                                      mlp_user, mlp_item)
    w1a = W1[:, :MLP_DIM].T
    w1b = W1[:, MLP_DIM:].T
    return _mlp_call(mlpu, mlpi, mfu, mfi,
                     w1a, w1b, b1.reshape(1, MLP_DIM),
                     W2.T, b2.reshape(1, MF_DIM),
                     Wp[:, :MF_DIM], Wp[:, MF_DIM:], bp.reshape(1, 1))
